# baseline (speedup 1.0000x reference)
import functools

import jax
import jax.numpy as jnp
from jax import lax
from jax.experimental import pallas as pl
from jax.experimental.pallas import tpu as pltpu
from jax.experimental.pallas import tpu_sc as plsc

N = 10000
E = 160000
NG = 64
NA = 128
H = 16
HP = lax.Precision.HIGHEST

NC = 2
NS = 16
NW = NC * NS
CHUNK = 128
NCHUNK = 40
EPW = CHUNK * NCHUNK
EPAD = EPW * NW
NPAD = 10240
STRIPE = NPAD // NS
DUMMY = N


def _stats_body(xe_ref, xn_ref, s1e_ref, s2e_ref, s1n_ref, ssn_ref):
    @pl.when(pl.program_id(0) == 0)
    def _():
        s1e_ref[...] = jnp.zeros_like(s1e_ref)
        s2e_ref[...] = jnp.zeros_like(s2e_ref)
        s1n_ref[...] = jnp.zeros_like(s1n_ref)
        ssn_ref[...] = jnp.zeros_like(ssn_ref)
    xe = xe_ref[...]
    s1e_ref[...] += jnp.sum(xe, axis=0, keepdims=True)
    s2e_ref[...] += lax.dot_general(xe, xe, (((0,), (0,)), ((), ())), precision=HP)
    xn = xn_ref[...]
    s1n_ref[...] += jnp.sum(xn, axis=0, keepdims=True)
    ssn_ref[...] += jnp.sum(xn * xn, axis=0, keepdims=True)


def _a_hn_body(xe_ref, m_ref, c_ref, xn_ref, mn_ref, cn_ref,
               a_ref, sa_ref, ga_ref, hn_ref):
    @pl.when(pl.program_id(0) == 0)
    def _():
        sa_ref[...] = jnp.zeros_like(sa_ref)
        ga_ref[...] = jnp.zeros_like(ga_ref)
    z = lax.dot_general(xe_ref[...], m_ref[...], (((1,), (0,)), ((), ())),
                        precision=HP) + c_ref[...]
    a = jnp.where(z >= 0, z, 0.8 * z)
    a_ref[...] = a
    sa_ref[...] += jnp.sum(a, axis=0, keepdims=True)
    ga_ref[...] += lax.dot_general(a, a, (((0,), (0,)), ((), ())), precision=HP)
    hn_ref[...] = lax.dot_general(xn_ref[...], mn_ref[...], (((1,), (0,)), ((), ())),
                                  precision=HP) + cn_ref[...]


def _msg_body(hs_ref, a_ref, rm_ref, tm_ref, b2_ref, d_ref, o_ref):
    hs = hs_ref[...]
    av = a_ref[...]
    rep = lax.dot_general(hs, rm_ref[...], (((1,), (0,)), ((), ())), precision=HP)
    til = lax.dot_general(av, tm_ref[...], (((1,), (0,)), ((), ())), precision=HP)
    op = rep * til
    msg = lax.dot_general(op, b2_ref[...], (((1,), (0,)), ((), ())), precision=HP)
    msg += lax.dot_general(hs, d_ref[...], (((1,), (0,)), ((), ())), precision=HP)
    o_ref[...] = msg


def _gru_body(a0_ref, a1_ref, d0_ref, d1_ref, h_ref, wih_ref, whh_ref,
              bih_ref, bhh_ref, o_ref):
    agg = a0_ref[...] + a1_ref[...]
    deg = d0_ref[:, 0:1] + d1_ref[:, 0:1]
    deg = jnp.maximum(deg, 1.0)
    x = agg / deg
    h = h_ref[...]
    gi = lax.dot_general(x, wih_ref[...], (((1,), (1,)), ((), ())),
                         precision=HP) + bih_ref[...]
    gh = lax.dot_general(h, whh_ref[...], (((1,), (1,)), ((), ())),
                         precision=HP) + bhh_ref[...]
    r = jax.nn.sigmoid(gi[:, 0:H] + gh[:, 0:H])
    z = jax.nn.sigmoid(gi[:, H:2 * H] + gh[:, H:2 * H])
    n = jnp.tanh(gi[:, 2 * H:3 * H] + r * gh[:, 2 * H:3 * H])
    o_ref[...] = (1.0 - z) * n + z * h


def _s2s_body(h_ref, n2g_ref, wih0_ref, wihr_ref, whh_ref, bih_ref, bhh_ref,
              bng_ref, bnb_ref, c1w_ref, c1b_ref, c2w_ref, c2b_ref, o_ref):
    h = h_ref[...]
    n2g = n2g_ref[...]
    gids = lax.broadcasted_iota(jnp.int32, (NG, N), 0)
    maskb = n2g == gids

    def step(t, carry):
        qs, hstack, cstack = carry
        inp = qs
        new_h = []
        new_c = []
        for l in range(4):
            wih = wih0_ref[...] if l == 0 else wihr_ref[(l - 1) * NG:l * NG, :]
            gates = (lax.dot_general(inp, wih, (((1,), (1,)), ((), ())),
                                     precision=HP)
                     + bih_ref[l:l + 1, :]
                     + lax.dot_general(hstack[l * NG:(l + 1) * NG, :],
                                       whh_ref[l * NG:(l + 1) * NG, :],
                                       (((1,), (1,)), ((), ())), precision=HP)
                     + bhh_ref[l:l + 1, :])
            gi_ = gates[:, 0:H]
            gf_ = gates[:, H:2 * H]
            gg_ = gates[:, 2 * H:3 * H]
            go_ = gates[:, 3 * H:4 * H]
            c = (jax.nn.sigmoid(gf_) * cstack[l * NG:(l + 1) * NG, :]
                 + jax.nn.sigmoid(gi_) * jnp.tanh(gg_))
            hc = jax.nn.sigmoid(go_) * jnp.tanh(c)
            new_h.append(hc)
            new_c.append(c)
            inp = hc
        q = inp
        emat = lax.dot_general(q, h, (((1,), (1,)), ((), ())), precision=HP)
        masked = jnp.where(maskb, emat, -1e30)
        emax = jnp.max(masked, axis=1, keepdims=True)
        ee = jnp.where(maskb, jnp.exp(masked - emax), 0.0)
        denom = jnp.maximum(jnp.sum(ee, axis=1, keepdims=True), 1e-30)
        alpha = ee / denom
        readout = lax.dot_general(alpha, h, (((1,), (0,)), ((), ())),
                                  precision=HP)
        return (jnp.concatenate([q, readout], axis=1),
                jnp.concatenate(new_h, axis=0), jnp.concatenate(new_c, axis=0))

    qs, _, _ = lax.fori_loop(
        0, 6, step,
        (jnp.zeros((NG, 2 * H), jnp.float32),
         jnp.zeros((4 * NG, H), jnp.float32),
         jnp.zeros((4 * NG, H), jnp.float32)))

    m = jnp.mean(qs, axis=0, keepdims=True)
    v = jnp.mean((qs - m) * (qs - m), axis=0, keepdims=True)
    qn_ = (qs - m) * lax.rsqrt(v + 1e-5) * bng_ref[...] + bnb_ref[...]
    o1 = lax.dot_general(qn_, c1w_ref[...], (((1,), (1,)), ((), ())),
                         precision=HP) + c1b_ref[...]
    o1 = jnp.where(o1 >= 0, o1, 0.1 * o1)
    o2 = jnp.sum(o1 * c2w_ref[...], axis=1, keepdims=True) + c2b_ref[...]
    o_ref[...] = jax.nn.sigmoid(o2)


_sc_cache = {}


def _sc_kernels():
    if _sc_cache:
        return _sc_cache
    mesh = plsc.VectorSubcoreMesh(core_axis_name="c", subcore_axis_name="s",
                                  num_cores=NC, num_subcores=NS)
    cparams = pltpu.CompilerParams(use_tc_tiling_on_sc=False)

    @functools.partial(
        pl.kernel,
        mesh=mesh,
        compiler_params=cparams,
        out_type=jax.ShapeDtypeStruct((EPAD, H), jnp.float32),
        scratch_types=[
            pltpu.VMEM((CHUNK,), jnp.int32),
            pltpu.VMEM((CHUNK, H), jnp.float32),
            pltpu.SemaphoreType.DMA,
        ],
    )
    def sc_gather(h_hbm, src_hbm, out_hbm, idx_v, rows_v, sem):
        wid = lax.axis_index("s") * NC + lax.axis_index("c")
        base = wid * EPW

        def body(c, carry):
            pltpu.sync_copy(src_hbm.at[wid, pl.ds(c * CHUNK, CHUNK)], idx_v)
            pltpu.async_copy(h_hbm.at[idx_v], rows_v, sem).wait()
            pltpu.sync_copy(rows_v, out_hbm.at[pl.ds(base + c * CHUNK, CHUNK)])
            return carry

        lax.fori_loop(0, NCHUNK, body, 0)

    @functools.partial(
        pl.kernel,
        mesh=mesh,
        compiler_params=cparams,
        out_type=jax.ShapeDtypeStruct((NC * NPAD, H), jnp.float32),
        scratch_types=[
            pltpu.VMEM((CHUNK,), jnp.int32),
            pltpu.VMEM((CHUNK, H), jnp.float32),
            pltpu.VMEM((STRIPE, H), jnp.float32),
            pltpu.VMEM_SHARED((NPAD, H), jnp.float32),
        ],
    )
    def sc_scatter(msg_hbm, dst_hbm, out_hbm, idx_v, rows_v, zbuf, shared):
        cid = lax.axis_index("c")
        sid = lax.axis_index("s")
        wid = sid * NC + cid
        base = wid * EPW

        def zr(i, carry):
            zbuf[i, :] = jnp.zeros((H,), jnp.float32)
            return carry

        lax.fori_loop(0, STRIPE, zr, 0)
        pltpu.sync_copy(zbuf, shared.at[pl.ds(sid * STRIPE, STRIPE)])
        plsc.subcore_barrier()

        def body(c, carry):
            pltpu.sync_copy(dst_hbm.at[wid, pl.ds(c * CHUNK, CHUNK)], idx_v)
            pltpu.sync_copy(msg_hbm.at[pl.ds(base + c * CHUNK, CHUNK)], rows_v)
            pltpu.sync_copy(rows_v, shared.at[idx_v], add=True)
            return carry

        lax.fori_loop(0, NCHUNK, body, 0)
        plsc.subcore_barrier()
        pltpu.sync_copy(shared.at[pl.ds(sid * STRIPE, STRIPE)],
                        out_hbm.at[pl.ds(cid * NPAD + sid * STRIPE, STRIPE)])

    @functools.partial(
        pl.kernel,
        mesh=mesh,
        compiler_params=cparams,
        out_type=[jax.ShapeDtypeStruct((NC * NPAD, H), jnp.float32),
                  jax.ShapeDtypeStruct((NC * NPAD, H), jnp.float32)],
        scratch_types=[
            pltpu.VMEM((CHUNK,), jnp.int32),
            pltpu.VMEM((CHUNK, H), jnp.float32),
            pltpu.VMEM((CHUNK, H), jnp.float32),
            pltpu.VMEM((STRIPE, H), jnp.float32),
            pltpu.VMEM_SHARED((NPAD, H), jnp.float32),
            pltpu.VMEM_SHARED((NPAD, H), jnp.float32),
        ],
    )
    def sc_scatter_deg(msg_hbm, dst_hbm, agg_hbm, deg_hbm, idx_v, rows_v,
                       ones_v, zbuf, shared_m, shared_d):
        cid = lax.axis_index("c")
        sid = lax.axis_index("s")
        wid = sid * NC + cid
        base = wid * EPW

        def zr(i, carry):
            zbuf[i, :] = jnp.zeros((H,), jnp.float32)
            return carry

        lax.fori_loop(0, STRIPE, zr, 0)

        def onr(i, carry):
            ones_v[i, :] = jnp.ones((H,), jnp.float32)
            return carry

        lax.fori_loop(0, CHUNK, onr, 0)
        pltpu.sync_copy(zbuf, shared_m.at[pl.ds(sid * STRIPE, STRIPE)])
        pltpu.sync_copy(zbuf, shared_d.at[pl.ds(sid * STRIPE, STRIPE)])
        plsc.subcore_barrier()

        def body(c, carry):
            pltpu.sync_copy(dst_hbm.at[wid, pl.ds(c * CHUNK, CHUNK)], idx_v)
            pltpu.sync_copy(msg_hbm.at[pl.ds(base + c * CHUNK, CHUNK)], rows_v)
            pltpu.sync_copy(rows_v, shared_m.at[idx_v], add=True)
            pltpu.sync_copy(ones_v, shared_d.at[idx_v], add=True)
            return carry

        lax.fori_loop(0, NCHUNK, body, 0)
        plsc.subcore_barrier()
        pltpu.sync_copy(shared_m.at[pl.ds(sid * STRIPE, STRIPE)],
                        agg_hbm.at[pl.ds(cid * NPAD + sid * STRIPE, STRIPE)])
        pltpu.sync_copy(shared_d.at[pl.ds(sid * STRIPE, STRIPE)],
                        deg_hbm.at[pl.ds(cid * NPAD + sid * STRIPE, STRIPE)])

    _sc_cache.update(gather=sc_gather, scatter=sc_scatter,
                     scatter_deg=sc_scatter_deg)
    return _sc_cache


def _stats(x_edge, x_node):
    nt = 25
    tbe = E // nt
    tbn = N // nt
    return pl.pallas_call(
        _stats_body,
        grid=(nt,),
        in_specs=[pl.BlockSpec((tbe, H), lambda i: (i, 0)),
                  pl.BlockSpec((tbn, NA), lambda i: (i, 0))],
        out_specs=[pl.BlockSpec((1, H), lambda i: (0, 0)),
                   pl.BlockSpec((H, H), lambda i: (0, 0)),
                   pl.BlockSpec((1, NA), lambda i: (0, 0)),
                   pl.BlockSpec((1, NA), lambda i: (0, 0))],
        out_shape=[jax.ShapeDtypeStruct((1, H), jnp.float32),
                   jax.ShapeDtypeStruct((H, H), jnp.float32),
                   jax.ShapeDtypeStruct((1, NA), jnp.float32),
                   jax.ShapeDtypeStruct((1, NA), jnp.float32)],
    )(x_edge, x_node)


def _a_hn(x_edge, m2, c2, x_node, mn, cn):
    nt = 25
    tbe = E // nt
    tbn = N // nt
    return pl.pallas_call(
        _a_hn_body,
        grid=(nt,),
        in_specs=[pl.BlockSpec((tbe, H), lambda i: (i, 0)),
                  pl.BlockSpec((H, H), lambda i: (0, 0)),
                  pl.BlockSpec((1, H), lambda i: (0, 0)),
                  pl.BlockSpec((tbn, NA), lambda i: (i, 0)),
                  pl.BlockSpec((NA, H), lambda i: (0, 0)),
                  pl.BlockSpec((1, H), lambda i: (0, 0))],
        out_specs=[pl.BlockSpec((tbe, H), lambda i: (i, 0)),
                   pl.BlockSpec((1, H), lambda i: (0, 0)),
                   pl.BlockSpec((H, H), lambda i: (0, 0)),
                   pl.BlockSpec((tbn, H), lambda i: (i, 0))],
        out_shape=[jax.ShapeDtypeStruct((E, H), jnp.float32),
                   jax.ShapeDtypeStruct((1, H), jnp.float32),
                   jax.ShapeDtypeStruct((H, H), jnp.float32),
                   jax.ShapeDtypeStruct((N, H), jnp.float32)],
    )(x_edge, m2, c2, x_node, mn, cn)


def _edge_msg(hsrc, a_unpadded, rmat, tmat, b2, dmat):
    nt = 40
    tb = EPAD // nt
    return pl.pallas_call(
        _msg_body,
        grid=(nt,),
        in_specs=[pl.BlockSpec((tb, H), lambda i: (i, 0)),
                  pl.BlockSpec((tb, H), lambda i: (i, 0)),
                  pl.BlockSpec((H, H * H), lambda i: (0, 0)),
                  pl.BlockSpec((H, H * H), lambda i: (0, 0)),
                  pl.BlockSpec((H * H, H), lambda i: (0, 0)),
                  pl.BlockSpec((H, H), lambda i: (0, 0))],
        out_specs=pl.BlockSpec((tb, H), lambda i: (i, 0)),
        out_shape=jax.ShapeDtypeStruct((EPAD, H), jnp.float32),
    )(hsrc, a_unpadded, rmat, tmat, b2, dmat)


def _gru(aggp, degp, h, wih, whh, bih, bhh):
    nt = 10
    tb = N // nt
    row = pl.BlockSpec((tb, H), lambda i: (i, 0))
    full = lambda s: pl.BlockSpec(s, lambda i: (0, 0))
    return pl.pallas_call(
        _gru_body,
        grid=(nt,),
        in_specs=[row, row, row, row, row,
                  full((3 * H, H)), full((3 * H, H)),
                  full((1, 3 * H)), full((1, 3 * H))],
        out_specs=row,
        out_shape=jax.ShapeDtypeStruct((N, H), jnp.float32),
    )(aggp[0:N], aggp[NPAD:NPAD + N], degp[0:N], degp[NPAD:NPAD + N],
      h, wih, whh, bih, bhh)


def _set2set(h, n2g, p):
    return pl.pallas_call(
        _s2s_body,
        out_shape=jax.ShapeDtypeStruct((NG, 1), jnp.float32),
    )(h, n2g,
      p['lstm_Wih0'],
      p['lstm_Wih_rest'].reshape(3 * 4 * H, H),
      p['lstm_Whh'].reshape(4 * 4 * H, H),
      p['lstm_bih'], p['lstm_bhh'],
      p['bn_o_g'].reshape(1, 2 * H), p['bn_o_b'].reshape(1, 2 * H),
      p['c1_W'], p['c1_b'].reshape(1, H),
      p['c2_W'].reshape(1, H), jnp.broadcast_to(p['c2_b'].reshape(1, 1), (NG, 1)))


def kernel(x_node, x_edge, edge_index, node2graph, params):
    p = params
    src = edge_index[0]
    dst = edge_index[1]

    s1e, s2e, s1n, ssn = _stats(x_edge, x_node)

    ef = float(E)
    mu_x = s1e[0] / ef
    cov = s2e / ef - jnp.outer(mu_x, mu_x)
    var_x = jnp.diag(cov)
    se = p['bn_e_g'] * lax.rsqrt(var_x + 1e-5)
    c0 = p['bn_e_b'] - mu_x * se
    m_he = se[:, None] * p['eemb_W'].T
    c_he = c0 @ p['eemb_W'].T + p['eemb_b']
    m1 = m_he @ p['en1_W'].T
    c1v = c_he @ p['en1_W'].T + p['en1_b']
    mean1 = mu_x @ m1 + c1v
    var1 = jnp.sum(m1 * (cov @ m1), axis=0)
    s1 = p['enbn1_g'] * lax.rsqrt(var1 + 1e-5)
    m2 = m1 * s1[None, :]
    c2v = ((c1v - mean1) * s1 + p['enbn1_b']).reshape(1, H)

    mu_n = s1n[0] / float(N)
    var_n = ssn[0] / float(N) - mu_n * mu_n
    sn = p['bn_n_g'] * lax.rsqrt(var_n + 1e-5)
    mn = sn[:, None] * p['nemb_W'].T
    cn = ((p['bn_n_b'] - mu_n * sn) @ p['nemb_W'].T + p['nemb_b']).reshape(1, H)

    a, sa, ga, hn = _a_hn(x_edge, m2, c2v, x_node, mn, cn)

    mean_a = sa[0] / ef
    cov_a = ga / ef - jnp.outer(mean_a, mean_a)
    mean2 = mean_a @ p['en2_W'].T + p['en2_b']
    var2 = jnp.sum((p['en2_W'] @ cov_a) * p['en2_W'], axis=1)
    s2 = p['enbn2_g'] * lax.rsqrt(var2 + 1e-5)
    t2 = p['enbn2_b'] - mean2 * s2
    w3 = p['en2_W'].reshape(H, H, H)
    ahat = w3 * s2.reshape(H, H)[:, :, None]
    b2 = jnp.transpose(ahat, (0, 2, 1)).reshape(H * H, H)
    dmat = (s2 * p['en2_b'] + t2).reshape(H, H)
    k_ids = jnp.arange(H * H, dtype=jnp.int32)
    rows = jnp.arange(H, dtype=jnp.int32)
    rmat = (k_ids[None, :] // H == rows[:, None]).astype(jnp.float32)
    tmat = (k_ids[None, :] % H == rows[:, None]).astype(jnp.float32)

    src_p = jnp.pad(src, (0, EPAD - E)).reshape(NW, EPW)
    dst_p = jnp.pad(dst, (0, EPAD - E), constant_values=DUMMY).reshape(NW, EPW)

    sc = _sc_kernels()

    wih = p['gru_Wih']
    whh = p['gru_Whh']
    bih = p['gru_bih'].reshape(1, 3 * H)
    bhh = p['gru_bhh'].reshape(1, 3 * H)

    hsrc = sc['gather'](hn, src_p)
    msg = _edge_msg(hsrc, a, rmat, tmat, b2, dmat)
    aggp, degp = sc['scatter_deg'](msg, dst_p)
    h = _gru(aggp, degp, hn, wih, whh, bih, bhh)

    hsrc = sc['gather'](h, src_p)
    msg = _edge_msg(hsrc, a, rmat, tmat, b2, dmat)
    aggp = sc['scatter'](msg, dst_p)
    h = _gru(aggp, degp, h, wih, whh, bih, bhh)

    return _set2set(h, node2graph.reshape(1, N), p)


# speedup vs baseline: 1.8315x; 1.8315x over previous
import functools

import jax
import jax.numpy as jnp
from jax import lax
from jax.experimental import pallas as pl
from jax.experimental.pallas import tpu as pltpu
from jax.experimental.pallas import tpu_sc as plsc

N = 10000
E = 160000
NG = 64
NA = 128
H = 16
HP = lax.Precision.HIGHEST
DP = lax.Precision.DEFAULT

NC = 2
NS = 16
NW = NC * NS
CHUNK = 128
NCHUNK = 40
EPW = CHUNK * NCHUNK
EPAD = EPW * NW
NPAD = 10240
STRIPE = NPAD // NS
DUMMY = N


def _stats_body(xe_ref, xn_ref, s1e_ref, s2e_ref, s1n_ref, ssn_ref):
    @pl.when(pl.program_id(0) == 0)
    def _():
        s1e_ref[...] = jnp.zeros_like(s1e_ref)
        s2e_ref[...] = jnp.zeros_like(s2e_ref)
        s1n_ref[...] = jnp.zeros_like(s1n_ref)
        ssn_ref[...] = jnp.zeros_like(ssn_ref)
    xe = xe_ref[...]
    s1e_ref[...] += jnp.sum(xe, axis=0, keepdims=True)
    s2e_ref[...] += lax.dot_general(xe, xe, (((0,), (0,)), ((), ())), precision=HP)
    xn = xn_ref[...]
    s1n_ref[...] += jnp.sum(xn, axis=0, keepdims=True)
    ssn_ref[...] += jnp.sum(xn * xn, axis=0, keepdims=True)


def _a_hn_body(xe_ref, m_ref, c_ref, xn_ref, mn_ref, cn_ref,
               a_ref, sa_ref, ga_ref, hn_ref):
    @pl.when(pl.program_id(0) == 0)
    def _():
        sa_ref[...] = jnp.zeros_like(sa_ref)
        ga_ref[...] = jnp.zeros_like(ga_ref)
    z = lax.dot_general(xe_ref[...], m_ref[...], (((1,), (0,)), ((), ())),
                        precision=HP) + c_ref[...]
    a = jnp.where(z >= 0, z, 0.8 * z)
    a_ref[...] = a
    sa_ref[...] += jnp.sum(a, axis=0, keepdims=True)
    ga_ref[...] += lax.dot_general(a, a, (((0,), (0,)), ((), ())), precision=HP)
    hn_ref[...] = lax.dot_general(xn_ref[...], mn_ref[...], (((1,), (0,)), ((), ())),
                                  precision=HP) + cn_ref[...]


def _msg_body(hs_ref, a_ref, rm_ref, tm_ref, b2_ref, d_ref, o_ref):
    hs = hs_ref[...]
    av = a_ref[...]
    rep = lax.dot_general(hs, rm_ref[...], (((1,), (0,)), ((), ())), precision=DP)
    til = lax.dot_general(av, tm_ref[...], (((1,), (0,)), ((), ())), precision=DP)
    op = rep * til
    msg = lax.dot_general(op, b2_ref[...], (((1,), (0,)), ((), ())), precision=DP)
    msg += lax.dot_general(hs, d_ref[...], (((1,), (0,)), ((), ())), precision=DP)
    o_ref[...] = msg


def _gru_body(a0_ref, a1_ref, d0_ref, d1_ref, h_ref, wih_ref, whh_ref,
              bih_ref, bhh_ref, o_ref):
    agg = a0_ref[...] + a1_ref[...]
    deg = d0_ref[:, 0:1] + d1_ref[:, 0:1]
    deg = jnp.maximum(deg, 1.0)
    x = agg / deg
    h = h_ref[...]
    gi = lax.dot_general(x, wih_ref[...], (((1,), (1,)), ((), ())),
                         precision=HP) + bih_ref[...]
    gh = lax.dot_general(h, whh_ref[...], (((1,), (1,)), ((), ())),
                         precision=HP) + bhh_ref[...]
    r = jax.nn.sigmoid(gi[:, 0:H] + gh[:, 0:H])
    z = jax.nn.sigmoid(gi[:, H:2 * H] + gh[:, H:2 * H])
    n = jnp.tanh(gi[:, 2 * H:3 * H] + r * gh[:, 2 * H:3 * H])
    o_ref[...] = (1.0 - z) * n + z * h


def _s2s_body(h_ref, n2g_ref, wih0_ref, wihr_ref, whh_ref, bih_ref, bhh_ref,
              bng_ref, bnb_ref, c1w_ref, c1b_ref, c2w_ref, c2b_ref, o_ref):
    h = h_ref[...]
    n2g = n2g_ref[...]
    gids = lax.broadcasted_iota(jnp.int32, (NG, N), 0)
    maskb = n2g == gids

    def step(t, carry):
        qs, hstack, cstack = carry
        inp = qs
        new_h = []
        new_c = []
        for l in range(4):
            wih = wih0_ref[...] if l == 0 else wihr_ref[(l - 1) * NG:l * NG, :]
            gates = (lax.dot_general(inp, wih, (((1,), (1,)), ((), ())),
                                     precision=HP)
                     + bih_ref[l:l + 1, :]
                     + lax.dot_general(hstack[l * NG:(l + 1) * NG, :],
                                       whh_ref[l * NG:(l + 1) * NG, :],
                                       (((1,), (1,)), ((), ())), precision=HP)
                     + bhh_ref[l:l + 1, :])
            gi_ = gates[:, 0:H]
            gf_ = gates[:, H:2 * H]
            gg_ = gates[:, 2 * H:3 * H]
            go_ = gates[:, 3 * H:4 * H]
            c = (jax.nn.sigmoid(gf_) * cstack[l * NG:(l + 1) * NG, :]
                 + jax.nn.sigmoid(gi_) * jnp.tanh(gg_))
            hc = jax.nn.sigmoid(go_) * jnp.tanh(c)
            new_h.append(hc)
            new_c.append(c)
            inp = hc
        q = inp
        emat = lax.dot_general(q, h, (((1,), (1,)), ((), ())), precision=HP)
        masked = jnp.where(maskb, emat, -1e30)
        emax = jnp.max(masked, axis=1, keepdims=True)
        ee = jnp.where(maskb, jnp.exp(masked - emax), 0.0)
        denom = jnp.maximum(jnp.sum(ee, axis=1, keepdims=True), 1e-30)
        alpha = ee / denom
        readout = lax.dot_general(alpha, h, (((1,), (0,)), ((), ())),
                                  precision=HP)
        return (jnp.concatenate([q, readout], axis=1),
                jnp.concatenate(new_h, axis=0), jnp.concatenate(new_c, axis=0))

    qs, _, _ = lax.fori_loop(
        0, 6, step,
        (jnp.zeros((NG, 2 * H), jnp.float32),
         jnp.zeros((4 * NG, H), jnp.float32),
         jnp.zeros((4 * NG, H), jnp.float32)))

    m = jnp.mean(qs, axis=0, keepdims=True)
    v = jnp.mean((qs - m) * (qs - m), axis=0, keepdims=True)
    qn_ = (qs - m) * lax.rsqrt(v + 1e-5) * bng_ref[...] + bnb_ref[...]
    o1 = lax.dot_general(qn_, c1w_ref[...], (((1,), (1,)), ((), ())),
                         precision=HP) + c1b_ref[...]
    o1 = jnp.where(o1 >= 0, o1, 0.1 * o1)
    o2 = jnp.sum(o1 * c2w_ref[...], axis=1, keepdims=True) + c2b_ref[...]
    o_ref[...] = jax.nn.sigmoid(o2)


_sc_cache = {}


def _sc_kernels():
    if _sc_cache:
        return _sc_cache
    mesh = plsc.VectorSubcoreMesh(core_axis_name="c", subcore_axis_name="s",
                                  num_cores=NC, num_subcores=NS)
    cparams = pltpu.CompilerParams(use_tc_tiling_on_sc=False)

    @functools.partial(
        pl.kernel,
        mesh=mesh,
        compiler_params=cparams,
        out_type=jax.ShapeDtypeStruct((EPAD, H), jnp.float32),
        scratch_types=[
            pltpu.VMEM((CHUNK,), jnp.int32),
            pltpu.VMEM((CHUNK, H), jnp.float32),
            pltpu.SemaphoreType.DMA,
        ],
    )
    def sc_gather(h_hbm, src_hbm, out_hbm, idx_v, rows_v, sem):
        wid = lax.axis_index("s") * NC + lax.axis_index("c")
        base = wid * EPW

        def body(c, carry):
            pltpu.sync_copy(src_hbm.at[wid, pl.ds(c * CHUNK, CHUNK)], idx_v)
            pltpu.async_copy(h_hbm.at[idx_v], rows_v, sem).wait()
            pltpu.sync_copy(rows_v, out_hbm.at[pl.ds(base + c * CHUNK, CHUNK)])
            return carry

        lax.fori_loop(0, NCHUNK, body, 0)

    @functools.partial(
        pl.kernel,
        mesh=mesh,
        compiler_params=cparams,
        out_type=jax.ShapeDtypeStruct((NC * NPAD, H), jnp.float32),
        scratch_types=[
            pltpu.VMEM((CHUNK,), jnp.int32),
            pltpu.VMEM((CHUNK, H), jnp.float32),
            pltpu.VMEM((STRIPE, H), jnp.float32),
            pltpu.VMEM_SHARED((NPAD, H), jnp.float32),
        ],
    )
    def sc_scatter(msg_hbm, dst_hbm, out_hbm, idx_v, rows_v, zbuf, shared):
        cid = lax.axis_index("c")
        sid = lax.axis_index("s")
        wid = sid * NC + cid
        base = wid * EPW

        def zr(i, carry):
            zbuf[i, :] = jnp.zeros((H,), jnp.float32)
            return carry

        lax.fori_loop(0, STRIPE, zr, 0)
        pltpu.sync_copy(zbuf, shared.at[pl.ds(sid * STRIPE, STRIPE)])
        plsc.subcore_barrier()

        def body(c, carry):
            pltpu.sync_copy(dst_hbm.at[wid, pl.ds(c * CHUNK, CHUNK)], idx_v)
            pltpu.sync_copy(msg_hbm.at[pl.ds(base + c * CHUNK, CHUNK)], rows_v)
            pltpu.sync_copy(rows_v, shared.at[idx_v], add=True)
            return carry

        lax.fori_loop(0, NCHUNK, body, 0)
        plsc.subcore_barrier()
        pltpu.sync_copy(shared.at[pl.ds(sid * STRIPE, STRIPE)],
                        out_hbm.at[pl.ds(cid * NPAD + sid * STRIPE, STRIPE)])

    @functools.partial(
        pl.kernel,
        mesh=mesh,
        compiler_params=cparams,
        out_type=[jax.ShapeDtypeStruct((NC * NPAD, H), jnp.float32),
                  jax.ShapeDtypeStruct((NC * NPAD, H), jnp.float32)],
        scratch_types=[
            pltpu.VMEM((CHUNK,), jnp.int32),
            pltpu.VMEM((CHUNK, H), jnp.float32),
            pltpu.VMEM((CHUNK, H), jnp.float32),
            pltpu.VMEM((STRIPE, H), jnp.float32),
            pltpu.VMEM_SHARED((NPAD, H), jnp.float32),
            pltpu.VMEM_SHARED((NPAD, H), jnp.float32),
        ],
    )
    def sc_scatter_deg(msg_hbm, dst_hbm, agg_hbm, deg_hbm, idx_v, rows_v,
                       ones_v, zbuf, shared_m, shared_d):
        cid = lax.axis_index("c")
        sid = lax.axis_index("s")
        wid = sid * NC + cid
        base = wid * EPW

        def zr(i, carry):
            zbuf[i, :] = jnp.zeros((H,), jnp.float32)
            return carry

        lax.fori_loop(0, STRIPE, zr, 0)

        def onr(i, carry):
            ones_v[i, :] = jnp.ones((H,), jnp.float32)
            return carry

        lax.fori_loop(0, CHUNK, onr, 0)
        pltpu.sync_copy(zbuf, shared_m.at[pl.ds(sid * STRIPE, STRIPE)])
        pltpu.sync_copy(zbuf, shared_d.at[pl.ds(sid * STRIPE, STRIPE)])
        plsc.subcore_barrier()

        def body(c, carry):
            pltpu.sync_copy(dst_hbm.at[wid, pl.ds(c * CHUNK, CHUNK)], idx_v)
            pltpu.sync_copy(msg_hbm.at[pl.ds(base + c * CHUNK, CHUNK)], rows_v)
            pltpu.sync_copy(rows_v, shared_m.at[idx_v], add=True)
            pltpu.sync_copy(ones_v, shared_d.at[idx_v], add=True)
            return carry

        lax.fori_loop(0, NCHUNK, body, 0)
        plsc.subcore_barrier()
        pltpu.sync_copy(shared_m.at[pl.ds(sid * STRIPE, STRIPE)],
                        agg_hbm.at[pl.ds(cid * NPAD + sid * STRIPE, STRIPE)])
        pltpu.sync_copy(shared_d.at[pl.ds(sid * STRIPE, STRIPE)],
                        deg_hbm.at[pl.ds(cid * NPAD + sid * STRIPE, STRIPE)])

    _sc_cache.update(gather=sc_gather, scatter=sc_scatter,
                     scatter_deg=sc_scatter_deg)
    return _sc_cache


def _stats(x_edge, x_node):
    nt = 25
    tbe = E // nt
    tbn = N // nt
    return pl.pallas_call(
        _stats_body,
        grid=(nt,),
        in_specs=[pl.BlockSpec((tbe, H), lambda i: (i, 0)),
                  pl.BlockSpec((tbn, NA), lambda i: (i, 0))],
        out_specs=[pl.BlockSpec((1, H), lambda i: (0, 0)),
                   pl.BlockSpec((H, H), lambda i: (0, 0)),
                   pl.BlockSpec((1, NA), lambda i: (0, 0)),
                   pl.BlockSpec((1, NA), lambda i: (0, 0))],
        out_shape=[jax.ShapeDtypeStruct((1, H), jnp.float32),
                   jax.ShapeDtypeStruct((H, H), jnp.float32),
                   jax.ShapeDtypeStruct((1, NA), jnp.float32),
                   jax.ShapeDtypeStruct((1, NA), jnp.float32)],
    )(x_edge, x_node)


def _a_hn(x_edge, m2, c2, x_node, mn, cn):
    nt = 25
    tbe = E // nt
    tbn = N // nt
    return pl.pallas_call(
        _a_hn_body,
        grid=(nt,),
        in_specs=[pl.BlockSpec((tbe, H), lambda i: (i, 0)),
                  pl.BlockSpec((H, H), lambda i: (0, 0)),
                  pl.BlockSpec((1, H), lambda i: (0, 0)),
                  pl.BlockSpec((tbn, NA), lambda i: (i, 0)),
                  pl.BlockSpec((NA, H), lambda i: (0, 0)),
                  pl.BlockSpec((1, H), lambda i: (0, 0))],
        out_specs=[pl.BlockSpec((tbe, H), lambda i: (i, 0)),
                   pl.BlockSpec((1, H), lambda i: (0, 0)),
                   pl.BlockSpec((H, H), lambda i: (0, 0)),
                   pl.BlockSpec((tbn, H), lambda i: (i, 0))],
        out_shape=[jax.ShapeDtypeStruct((E, H), jnp.float32),
                   jax.ShapeDtypeStruct((1, H), jnp.float32),
                   jax.ShapeDtypeStruct((H, H), jnp.float32),
                   jax.ShapeDtypeStruct((N, H), jnp.float32)],
    )(x_edge, m2, c2, x_node, mn, cn)


def _edge_msg(hsrc, a_unpadded, rmat, tmat, b2, dmat):
    nt = 40
    tb = EPAD // nt
    return pl.pallas_call(
        _msg_body,
        grid=(nt,),
        in_specs=[pl.BlockSpec((tb, H), lambda i: (i, 0)),
                  pl.BlockSpec((tb, H), lambda i: (i, 0)),
                  pl.BlockSpec((H, H * H), lambda i: (0, 0)),
                  pl.BlockSpec((H, H * H), lambda i: (0, 0)),
                  pl.BlockSpec((H * H, H), lambda i: (0, 0)),
                  pl.BlockSpec((H, H), lambda i: (0, 0))],
        out_specs=pl.BlockSpec((tb, H), lambda i: (i, 0)),
        out_shape=jax.ShapeDtypeStruct((EPAD, H), jnp.float32),
    )(hsrc, a_unpadded, rmat, tmat, b2, dmat)


def _gru(aggp, degp, h, wih, whh, bih, bhh):
    nt = 10
    tb = N // nt
    row = pl.BlockSpec((tb, H), lambda i: (i, 0))
    full = lambda s: pl.BlockSpec(s, lambda i: (0, 0))
    return pl.pallas_call(
        _gru_body,
        grid=(nt,),
        in_specs=[row, row, row, row, row,
                  full((3 * H, H)), full((3 * H, H)),
                  full((1, 3 * H)), full((1, 3 * H))],
        out_specs=row,
        out_shape=jax.ShapeDtypeStruct((N, H), jnp.float32),
    )(aggp[0:N], aggp[NPAD:NPAD + N], degp[0:N], degp[NPAD:NPAD + N],
      h, wih, whh, bih, bhh)


def _set2set(h, n2g, p):
    return pl.pallas_call(
        _s2s_body,
        out_shape=jax.ShapeDtypeStruct((NG, 1), jnp.float32),
    )(h, n2g,
      p['lstm_Wih0'],
      p['lstm_Wih_rest'].reshape(3 * 4 * H, H),
      p['lstm_Whh'].reshape(4 * 4 * H, H),
      p['lstm_bih'], p['lstm_bhh'],
      p['bn_o_g'].reshape(1, 2 * H), p['bn_o_b'].reshape(1, 2 * H),
      p['c1_W'], p['c1_b'].reshape(1, H),
      p['c2_W'].reshape(1, H), jnp.broadcast_to(p['c2_b'].reshape(1, 1), (NG, 1)))


def kernel(x_node, x_edge, edge_index, node2graph, params):
    p = params
    src = edge_index[0]
    dst = edge_index[1]

    s1e, s2e, s1n, ssn = _stats(x_edge, x_node)

    ef = float(E)
    mu_x = s1e[0] / ef
    cov = s2e / ef - jnp.outer(mu_x, mu_x)
    var_x = jnp.diag(cov)
    se = p['bn_e_g'] * lax.rsqrt(var_x + 1e-5)
    c0 = p['bn_e_b'] - mu_x * se
    m_he = se[:, None] * p['eemb_W'].T
    c_he = c0 @ p['eemb_W'].T + p['eemb_b']
    m1 = m_he @ p['en1_W'].T
    c1v = c_he @ p['en1_W'].T + p['en1_b']
    mean1 = mu_x @ m1 + c1v
    var1 = jnp.sum(m1 * (cov @ m1), axis=0)
    s1 = p['enbn1_g'] * lax.rsqrt(var1 + 1e-5)
    m2 = m1 * s1[None, :]
    c2v = ((c1v - mean1) * s1 + p['enbn1_b']).reshape(1, H)

    mu_n = s1n[0] / float(N)
    var_n = ssn[0] / float(N) - mu_n * mu_n
    sn = p['bn_n_g'] * lax.rsqrt(var_n + 1e-5)
    mn = sn[:, None] * p['nemb_W'].T
    cn = ((p['bn_n_b'] - mu_n * sn) @ p['nemb_W'].T + p['nemb_b']).reshape(1, H)

    a, sa, ga, hn = _a_hn(x_edge, m2, c2v, x_node, mn, cn)

    mean_a = sa[0] / ef
    cov_a = ga / ef - jnp.outer(mean_a, mean_a)
    mean2 = mean_a @ p['en2_W'].T + p['en2_b']
    var2 = jnp.sum((p['en2_W'] @ cov_a) * p['en2_W'], axis=1)
    s2 = p['enbn2_g'] * lax.rsqrt(var2 + 1e-5)
    t2 = p['enbn2_b'] - mean2 * s2
    w3 = p['en2_W'].reshape(H, H, H)
    ahat = w3 * s2.reshape(H, H)[:, :, None]
    b2 = jnp.transpose(ahat, (0, 2, 1)).reshape(H * H, H)
    dmat = (s2 * p['en2_b'] + t2).reshape(H, H)
    k_ids = jnp.arange(H * H, dtype=jnp.int32)
    rows = jnp.arange(H, dtype=jnp.int32)
    rmat = (k_ids[None, :] // H == rows[:, None]).astype(jnp.float32)
    tmat = (k_ids[None, :] % H == rows[:, None]).astype(jnp.float32)

    src_p = jnp.pad(src, (0, EPAD - E)).reshape(NW, EPW)
    dst_p = jnp.pad(dst, (0, EPAD - E), constant_values=DUMMY).reshape(NW, EPW)

    sc = _sc_kernels()

    wih = p['gru_Wih']
    whh = p['gru_Whh']
    bih = p['gru_bih'].reshape(1, 3 * H)
    bhh = p['gru_bhh'].reshape(1, 3 * H)

    hsrc = sc['gather'](hn, src_p)
    msg = _edge_msg(hsrc, a, rmat, tmat, b2, dmat)
    aggp, degp = sc['scatter_deg'](msg, dst_p)
    h = _gru(aggp, degp, hn, wih, whh, bih, bhh)

    hsrc = sc['gather'](h, src_p)
    msg = _edge_msg(hsrc, a, rmat, tmat, b2, dmat)
    aggp = sc['scatter'](msg, dst_p)
    h = _gru(aggp, degp, h, wih, whh, bih, bhh)

    return _set2set(h, node2graph.reshape(1, N), p)


# speedup vs baseline: 2.0870x; 1.1395x over previous
import functools

import jax
import jax.numpy as jnp
from jax import lax
from jax.experimental import pallas as pl
from jax.experimental.pallas import tpu as pltpu
from jax.experimental.pallas import tpu_sc as plsc

N = 10000
E = 160000
NG = 64
NA = 128
H = 16
HP = lax.Precision.HIGHEST
DP = lax.Precision.DEFAULT

NC = 2
NS = 16
NW = NC * NS
CHUNK = 128
NCHUNK = 40
EPW = CHUNK * NCHUNK
EPAD = EPW * NW
NPAD = 10240
STRIPE = NPAD // NS
DUMMY = N


def _stats_body(xe_ref, xn_ref, s1e_ref, s2e_ref, s1n_ref, ssn_ref):
    @pl.when(pl.program_id(0) == 0)
    def _():
        s1e_ref[...] = jnp.zeros_like(s1e_ref)
        s2e_ref[...] = jnp.zeros_like(s2e_ref)
        s1n_ref[...] = jnp.zeros_like(s1n_ref)
        ssn_ref[...] = jnp.zeros_like(ssn_ref)
    xe = xe_ref[...]
    s1e_ref[...] += jnp.sum(xe, axis=0, keepdims=True)
    s2e_ref[...] += lax.dot_general(xe, xe, (((0,), (0,)), ((), ())), precision=HP)
    xn = xn_ref[...]
    s1n_ref[...] += jnp.sum(xn, axis=0, keepdims=True)
    ssn_ref[...] += jnp.sum(xn * xn, axis=0, keepdims=True)


def _a_hn_body(xe_ref, m_ref, c_ref, xn_ref, mn_ref, cn_ref,
               a_ref, sa_ref, ga_ref, hn_ref):
    @pl.when(pl.program_id(0) == 0)
    def _():
        sa_ref[...] = jnp.zeros_like(sa_ref)
        ga_ref[...] = jnp.zeros_like(ga_ref)
    z = lax.dot_general(xe_ref[...], m_ref[...], (((1,), (0,)), ((), ())),
                        precision=HP) + c_ref[...]
    a = jnp.where(z >= 0, z, 0.8 * z)
    a_ref[...] = a
    sa_ref[...] += jnp.sum(a, axis=0, keepdims=True)
    ga_ref[...] += lax.dot_general(a, a, (((0,), (0,)), ((), ())), precision=HP)
    hn_ref[...] = lax.dot_general(xn_ref[...], mn_ref[...], (((1,), (0,)), ((), ())),
                                  precision=HP) + cn_ref[...]


def _msg_body(hs_ref, a_ref, rm_ref, tm_ref, b2_ref, d_ref, o_ref):
    hs = hs_ref[...]
    av = a_ref[...]
    rep = lax.dot_general(hs, rm_ref[...], (((1,), (0,)), ((), ())), precision=DP)
    til = lax.dot_general(av, tm_ref[...], (((1,), (0,)), ((), ())), precision=DP)
    op = rep * til
    msg = lax.dot_general(op, b2_ref[...], (((1,), (0,)), ((), ())), precision=DP)
    msg += lax.dot_general(hs, d_ref[...], (((1,), (0,)), ((), ())), precision=DP)
    o_ref[...] = msg


def _gru_body(a0_ref, a1_ref, d0_ref, d1_ref, h_ref, wih_ref, whh_ref,
              bih_ref, bhh_ref, o_ref):
    agg = a0_ref[...] + a1_ref[...]
    deg = d0_ref[:, 0:1] + d1_ref[:, 0:1]
    deg = jnp.maximum(deg, 1.0)
    x = agg / deg
    h = h_ref[...]
    gi = lax.dot_general(x, wih_ref[...], (((1,), (1,)), ((), ())),
                         precision=HP) + bih_ref[...]
    gh = lax.dot_general(h, whh_ref[...], (((1,), (1,)), ((), ())),
                         precision=HP) + bhh_ref[...]
    r = jax.nn.sigmoid(gi[:, 0:H] + gh[:, 0:H])
    z = jax.nn.sigmoid(gi[:, H:2 * H] + gh[:, H:2 * H])
    n = jnp.tanh(gi[:, 2 * H:3 * H] + r * gh[:, 2 * H:3 * H])
    o_ref[...] = (1.0 - z) * n + z * h


def _s2s_body(h_ref, n2g_ref, wih0_ref, wihr_ref, whh_ref, bih_ref, bhh_ref,
              bng_ref, bnb_ref, c1w_ref, c1b_ref, c2w_ref, c2b_ref, o_ref):
    h = h_ref[...]
    n2g = n2g_ref[...]
    gids = lax.broadcasted_iota(jnp.int32, (NG, N), 0)
    maskb = n2g == gids

    def step(t, carry):
        qs, hstack, cstack = carry
        inp = qs
        new_h = []
        new_c = []
        for l in range(4):
            wih = wih0_ref[...] if l == 0 else wihr_ref[(l - 1) * NG:l * NG, :]
            gates = (lax.dot_general(inp, wih, (((1,), (1,)), ((), ())),
                                     precision=HP)
                     + bih_ref[l:l + 1, :]
                     + lax.dot_general(hstack[l * NG:(l + 1) * NG, :],
                                       whh_ref[l * NG:(l + 1) * NG, :],
                                       (((1,), (1,)), ((), ())), precision=HP)
                     + bhh_ref[l:l + 1, :])
            gi_ = gates[:, 0:H]
            gf_ = gates[:, H:2 * H]
            gg_ = gates[:, 2 * H:3 * H]
            go_ = gates[:, 3 * H:4 * H]
            c = (jax.nn.sigmoid(gf_) * cstack[l * NG:(l + 1) * NG, :]
                 + jax.nn.sigmoid(gi_) * jnp.tanh(gg_))
            hc = jax.nn.sigmoid(go_) * jnp.tanh(c)
            new_h.append(hc)
            new_c.append(c)
            inp = hc
        q = inp
        emat = lax.dot_general(q, h, (((1,), (1,)), ((), ())), precision=HP)
        masked = jnp.where(maskb, emat, -1e30)
        emax = jnp.max(masked, axis=1, keepdims=True)
        ee = jnp.where(maskb, jnp.exp(masked - emax), 0.0)
        denom = jnp.maximum(jnp.sum(ee, axis=1, keepdims=True), 1e-30)
        alpha = ee / denom
        readout = lax.dot_general(alpha, h, (((1,), (0,)), ((), ())),
                                  precision=HP)
        return (jnp.concatenate([q, readout], axis=1),
                jnp.concatenate(new_h, axis=0), jnp.concatenate(new_c, axis=0))

    qs, _, _ = lax.fori_loop(
        0, 6, step,
        (jnp.zeros((NG, 2 * H), jnp.float32),
         jnp.zeros((4 * NG, H), jnp.float32),
         jnp.zeros((4 * NG, H), jnp.float32)))

    m = jnp.mean(qs, axis=0, keepdims=True)
    v = jnp.mean((qs - m) * (qs - m), axis=0, keepdims=True)
    qn_ = (qs - m) * lax.rsqrt(v + 1e-5) * bng_ref[...] + bnb_ref[...]
    o1 = lax.dot_general(qn_, c1w_ref[...], (((1,), (1,)), ((), ())),
                         precision=HP) + c1b_ref[...]
    o1 = jnp.where(o1 >= 0, o1, 0.1 * o1)
    o2 = jnp.sum(o1 * c2w_ref[...], axis=1, keepdims=True) + c2b_ref[...]
    o_ref[...] = jax.nn.sigmoid(o2)


_sc_cache = {}


def _sc_kernels():
    if _sc_cache:
        return _sc_cache
    mesh = plsc.VectorSubcoreMesh(core_axis_name="c", subcore_axis_name="s",
                                  num_cores=NC, num_subcores=NS)
    cparams = pltpu.CompilerParams(use_tc_tiling_on_sc=False)

    @functools.partial(
        pl.kernel,
        mesh=mesh,
        compiler_params=cparams,
        out_type=jax.ShapeDtypeStruct((EPAD, H), jnp.float32),
        scratch_types=[
            pltpu.VMEM((NCHUNK, CHUNK), jnp.int32),
            pltpu.VMEM((EPW, H), jnp.float32),
            pltpu.SemaphoreType.DMA,
        ],
    )
    def sc_gather(h_hbm, src_hbm, out_hbm, idx_v, rows_v, sem):
        wid = lax.axis_index("s") * NC + lax.axis_index("c")
        base = wid * EPW
        pltpu.sync_copy(src_hbm.at[wid], idx_v)
        for g in range(0, NCHUNK, 20):
            fires = [
                pltpu.async_copy(h_hbm.at[idx_v.at[c]],
                                 rows_v.at[pl.ds(c * CHUNK, CHUNK)], sem)
                for c in range(g, g + 20)
            ]
            for d in fires:
                d.wait()
        pltpu.sync_copy(rows_v, out_hbm.at[pl.ds(base, EPW)])

    @functools.partial(
        pl.kernel,
        mesh=mesh,
        compiler_params=cparams,
        out_type=jax.ShapeDtypeStruct((NC * NPAD, H), jnp.float32),
        scratch_types=[
            pltpu.VMEM((NCHUNK, CHUNK), jnp.int32),
            pltpu.VMEM((EPW, H), jnp.float32),
            pltpu.VMEM((STRIPE, H), jnp.float32),
            pltpu.SemaphoreType.DMA,
            pltpu.VMEM_SHARED((NPAD, H), jnp.float32),
        ],
    )
    def sc_scatter(msg_hbm, dst_hbm, out_hbm, idx_v, rows_v, zbuf, sem, shared):
        cid = lax.axis_index("c")
        sid = lax.axis_index("s")
        wid = sid * NC + cid
        base = wid * EPW

        def zr(i, carry):
            zbuf[i, :] = jnp.zeros((H,), jnp.float32)
            return carry

        lax.fori_loop(0, STRIPE, zr, 0)
        pltpu.sync_copy(zbuf, shared.at[pl.ds(sid * STRIPE, STRIPE)])
        pltpu.sync_copy(dst_hbm.at[wid], idx_v)
        pltpu.sync_copy(msg_hbm.at[pl.ds(base, EPW)], rows_v)
        plsc.subcore_barrier()
        for g in range(0, NCHUNK, 20):
            fires = [
                pltpu.async_copy(rows_v.at[pl.ds(c * CHUNK, CHUNK)],
                                 shared.at[idx_v.at[c]], sem, add=True)
                for c in range(g, g + 20)
            ]
            for d in fires:
                d.wait()
        plsc.subcore_barrier()
        pltpu.sync_copy(shared.at[pl.ds(sid * STRIPE, STRIPE)],
                        out_hbm.at[pl.ds(cid * NPAD + sid * STRIPE, STRIPE)])

    @functools.partial(
        pl.kernel,
        mesh=mesh,
        compiler_params=cparams,
        out_type=[jax.ShapeDtypeStruct((NC * NPAD, H), jnp.float32),
                  jax.ShapeDtypeStruct((NC * NPAD, H), jnp.float32)],
        scratch_types=[
            pltpu.VMEM((NCHUNK, CHUNK), jnp.int32),
            pltpu.VMEM((EPW, H), jnp.float32),
            pltpu.VMEM((CHUNK, H), jnp.float32),
            pltpu.VMEM((STRIPE, H), jnp.float32),
            pltpu.SemaphoreType.DMA,
            pltpu.VMEM_SHARED((NPAD, H), jnp.float32),
            pltpu.VMEM_SHARED((NPAD, H), jnp.float32),
        ],
    )
    def sc_scatter_deg(msg_hbm, dst_hbm, agg_hbm, deg_hbm, idx_v, rows_v,
                       ones_v, zbuf, sem, shared_m, shared_d):
        cid = lax.axis_index("c")
        sid = lax.axis_index("s")
        wid = sid * NC + cid
        base = wid * EPW

        def zr(i, carry):
            zbuf[i, :] = jnp.zeros((H,), jnp.float32)
            return carry

        lax.fori_loop(0, STRIPE, zr, 0)

        def onr(i, carry):
            ones_v[i, :] = jnp.ones((H,), jnp.float32)
            return carry

        lax.fori_loop(0, CHUNK, onr, 0)
        pltpu.sync_copy(zbuf, shared_m.at[pl.ds(sid * STRIPE, STRIPE)])
        pltpu.sync_copy(zbuf, shared_d.at[pl.ds(sid * STRIPE, STRIPE)])
        pltpu.sync_copy(dst_hbm.at[wid], idx_v)
        pltpu.sync_copy(msg_hbm.at[pl.ds(base, EPW)], rows_v)
        plsc.subcore_barrier()
        for g in range(0, NCHUNK, 10):
            fires = []
            for c in range(g, g + 10):
                fires.append(
                    pltpu.async_copy(rows_v.at[pl.ds(c * CHUNK, CHUNK)],
                                     shared_m.at[idx_v.at[c]], sem, add=True))
                fires.append(
                    pltpu.async_copy(ones_v, shared_d.at[idx_v.at[c]], sem,
                                     add=True))
            for d in fires:
                d.wait()
        plsc.subcore_barrier()
        pltpu.sync_copy(shared_m.at[pl.ds(sid * STRIPE, STRIPE)],
                        agg_hbm.at[pl.ds(cid * NPAD + sid * STRIPE, STRIPE)])
        pltpu.sync_copy(shared_d.at[pl.ds(sid * STRIPE, STRIPE)],
                        deg_hbm.at[pl.ds(cid * NPAD + sid * STRIPE, STRIPE)])

    _sc_cache.update(gather=sc_gather, scatter=sc_scatter,
                     scatter_deg=sc_scatter_deg)
    return _sc_cache


def _stats(x_edge, x_node):
    nt = 25
    tbe = E // nt
    tbn = N // nt
    return pl.pallas_call(
        _stats_body,
        grid=(nt,),
        in_specs=[pl.BlockSpec((tbe, H), lambda i: (i, 0)),
                  pl.BlockSpec((tbn, NA), lambda i: (i, 0))],
        out_specs=[pl.BlockSpec((1, H), lambda i: (0, 0)),
                   pl.BlockSpec((H, H), lambda i: (0, 0)),
                   pl.BlockSpec((1, NA), lambda i: (0, 0)),
                   pl.BlockSpec((1, NA), lambda i: (0, 0))],
        out_shape=[jax.ShapeDtypeStruct((1, H), jnp.float32),
                   jax.ShapeDtypeStruct((H, H), jnp.float32),
                   jax.ShapeDtypeStruct((1, NA), jnp.float32),
                   jax.ShapeDtypeStruct((1, NA), jnp.float32)],
    )(x_edge, x_node)


def _a_hn(x_edge, m2, c2, x_node, mn, cn):
    nt = 25
    tbe = E // nt
    tbn = N // nt
    return pl.pallas_call(
        _a_hn_body,
        grid=(nt,),
        in_specs=[pl.BlockSpec((tbe, H), lambda i: (i, 0)),
                  pl.BlockSpec((H, H), lambda i: (0, 0)),
                  pl.BlockSpec((1, H), lambda i: (0, 0)),
                  pl.BlockSpec((tbn, NA), lambda i: (i, 0)),
                  pl.BlockSpec((NA, H), lambda i: (0, 0)),
                  pl.BlockSpec((1, H), lambda i: (0, 0))],
        out_specs=[pl.BlockSpec((tbe, H), lambda i: (i, 0)),
                   pl.BlockSpec((1, H), lambda i: (0, 0)),
                   pl.BlockSpec((H, H), lambda i: (0, 0)),
                   pl.BlockSpec((tbn, H), lambda i: (i, 0))],
        out_shape=[jax.ShapeDtypeStruct((E, H), jnp.float32),
                   jax.ShapeDtypeStruct((1, H), jnp.float32),
                   jax.ShapeDtypeStruct((H, H), jnp.float32),
                   jax.ShapeDtypeStruct((N, H), jnp.float32)],
    )(x_edge, m2, c2, x_node, mn, cn)


def _edge_msg(hsrc, a_unpadded, rmat, tmat, b2, dmat):
    nt = 40
    tb = EPAD // nt
    return pl.pallas_call(
        _msg_body,
        grid=(nt,),
        in_specs=[pl.BlockSpec((tb, H), lambda i: (i, 0)),
                  pl.BlockSpec((tb, H), lambda i: (i, 0)),
                  pl.BlockSpec((H, H * H), lambda i: (0, 0)),
                  pl.BlockSpec((H, H * H), lambda i: (0, 0)),
                  pl.BlockSpec((H * H, H), lambda i: (0, 0)),
                  pl.BlockSpec((H, H), lambda i: (0, 0))],
        out_specs=pl.BlockSpec((tb, H), lambda i: (i, 0)),
        out_shape=jax.ShapeDtypeStruct((EPAD, H), jnp.float32),
    )(hsrc, a_unpadded, rmat, tmat, b2, dmat)


def _gru(aggp, degp, h, wih, whh, bih, bhh):
    nt = 10
    tb = N // nt
    row = pl.BlockSpec((tb, H), lambda i: (i, 0))
    full = lambda s: pl.BlockSpec(s, lambda i: (0, 0))
    return pl.pallas_call(
        _gru_body,
        grid=(nt,),
        in_specs=[row, row, row, row, row,
                  full((3 * H, H)), full((3 * H, H)),
                  full((1, 3 * H)), full((1, 3 * H))],
        out_specs=row,
        out_shape=jax.ShapeDtypeStruct((N, H), jnp.float32),
    )(aggp[0:N], aggp[NPAD:NPAD + N], degp[0:N], degp[NPAD:NPAD + N],
      h, wih, whh, bih, bhh)


def _set2set(h, n2g, p):
    return pl.pallas_call(
        _s2s_body,
        out_shape=jax.ShapeDtypeStruct((NG, 1), jnp.float32),
    )(h, n2g,
      p['lstm_Wih0'],
      p['lstm_Wih_rest'].reshape(3 * 4 * H, H),
      p['lstm_Whh'].reshape(4 * 4 * H, H),
      p['lstm_bih'], p['lstm_bhh'],
      p['bn_o_g'].reshape(1, 2 * H), p['bn_o_b'].reshape(1, 2 * H),
      p['c1_W'], p['c1_b'].reshape(1, H),
      p['c2_W'].reshape(1, H), jnp.broadcast_to(p['c2_b'].reshape(1, 1), (NG, 1)))


def kernel(x_node, x_edge, edge_index, node2graph, params):
    p = params
    src = edge_index[0]
    dst = edge_index[1]

    s1e, s2e, s1n, ssn = _stats(x_edge, x_node)

    ef = float(E)
    mu_x = s1e[0] / ef
    cov = s2e / ef - jnp.outer(mu_x, mu_x)
    var_x = jnp.diag(cov)
    se = p['bn_e_g'] * lax.rsqrt(var_x + 1e-5)
    c0 = p['bn_e_b'] - mu_x * se
    m_he = se[:, None] * p['eemb_W'].T
    c_he = c0 @ p['eemb_W'].T + p['eemb_b']
    m1 = m_he @ p['en1_W'].T
    c1v = c_he @ p['en1_W'].T + p['en1_b']
    mean1 = mu_x @ m1 + c1v
    var1 = jnp.sum(m1 * (cov @ m1), axis=0)
    s1 = p['enbn1_g'] * lax.rsqrt(var1 + 1e-5)
    m2 = m1 * s1[None, :]
    c2v = ((c1v - mean1) * s1 + p['enbn1_b']).reshape(1, H)

    mu_n = s1n[0] / float(N)
    var_n = ssn[0] / float(N) - mu_n * mu_n
    sn = p['bn_n_g'] * lax.rsqrt(var_n + 1e-5)
    mn = sn[:, None] * p['nemb_W'].T
    cn = ((p['bn_n_b'] - mu_n * sn) @ p['nemb_W'].T + p['nemb_b']).reshape(1, H)

    a, sa, ga, hn = _a_hn(x_edge, m2, c2v, x_node, mn, cn)

    mean_a = sa[0] / ef
    cov_a = ga / ef - jnp.outer(mean_a, mean_a)
    mean2 = mean_a @ p['en2_W'].T + p['en2_b']
    var2 = jnp.sum((p['en2_W'] @ cov_a) * p['en2_W'], axis=1)
    s2 = p['enbn2_g'] * lax.rsqrt(var2 + 1e-5)
    t2 = p['enbn2_b'] - mean2 * s2
    w3 = p['en2_W'].reshape(H, H, H)
    ahat = w3 * s2.reshape(H, H)[:, :, None]
    b2 = jnp.transpose(ahat, (0, 2, 1)).reshape(H * H, H)
    dmat = (s2 * p['en2_b'] + t2).reshape(H, H)
    k_ids = jnp.arange(H * H, dtype=jnp.int32)
    rows = jnp.arange(H, dtype=jnp.int32)
    rmat = (k_ids[None, :] // H == rows[:, None]).astype(jnp.float32)
    tmat = (k_ids[None, :] % H == rows[:, None]).astype(jnp.float32)

    src_p = jnp.pad(src, (0, EPAD - E)).reshape(NW, NCHUNK, CHUNK)
    dst_p = jnp.pad(dst, (0, EPAD - E), constant_values=DUMMY).reshape(NW, NCHUNK, CHUNK)

    sc = _sc_kernels()

    wih = p['gru_Wih']
    whh = p['gru_Whh']
    bih = p['gru_bih'].reshape(1, 3 * H)
    bhh = p['gru_bhh'].reshape(1, 3 * H)

    hsrc = sc['gather'](hn, src_p)
    msg = _edge_msg(hsrc, a, rmat, tmat, b2, dmat)
    aggp, degp = sc['scatter_deg'](msg, dst_p)
    h = _gru(aggp, degp, hn, wih, whh, bih, bhh)

    hsrc = sc['gather'](h, src_p)
    msg = _edge_msg(hsrc, a, rmat, tmat, b2, dmat)
    aggp = sc['scatter'](msg, dst_p)
    h = _gru(aggp, degp, h, wih, whh, bih, bhh)

    return _set2set(h, node2graph.reshape(1, N), p)


# speedup vs baseline: 2.3666x; 1.1340x over previous
import functools

import jax
import jax.numpy as jnp
from jax import lax
from jax.experimental import pallas as pl
from jax.experimental.pallas import tpu as pltpu
from jax.experimental.pallas import tpu_sc as plsc

N = 10000
E = 160000
NG = 64
NA = 128
H = 16
HP = lax.Precision.HIGHEST
DP = lax.Precision.DEFAULT

NC = 2
NS = 16
NW = NC * NS
CHUNK = 128
NCHUNK = 40
EPW = CHUNK * NCHUNK
EPAD = EPW * NW
NPAD = 10240
STRIPE = NPAD // NS
DUMMY = N


def _stats_body(xe_ref, xn_ref, s1e_ref, s2e_ref, s1n_ref, ssn_ref):
    @pl.when(pl.program_id(0) == 0)
    def _():
        s1e_ref[...] = jnp.zeros_like(s1e_ref)
        s2e_ref[...] = jnp.zeros_like(s2e_ref)
        s1n_ref[...] = jnp.zeros_like(s1n_ref)
        ssn_ref[...] = jnp.zeros_like(ssn_ref)
    xe = xe_ref[...]
    s1e_ref[...] += jnp.sum(xe, axis=0, keepdims=True)
    s2e_ref[...] += lax.dot_general(xe, xe, (((0,), (0,)), ((), ())), precision=HP)
    xn = xn_ref[...]
    s1n_ref[...] += jnp.sum(xn, axis=0, keepdims=True)
    ssn_ref[...] += jnp.sum(xn * xn, axis=0, keepdims=True)


def _a_hn_body(xe_ref, m_ref, c_ref, xn_ref, mn_ref, cn_ref,
               a_ref, sa_ref, ga_ref, hn_ref):
    @pl.when(pl.program_id(0) == 0)
    def _():
        sa_ref[...] = jnp.zeros_like(sa_ref)
        ga_ref[...] = jnp.zeros_like(ga_ref)
    z = lax.dot_general(xe_ref[...], m_ref[...], (((1,), (0,)), ((), ())),
                        precision=DP) + c_ref[...]
    a = jnp.where(z >= 0, z, 0.8 * z)
    a_ref[...] = a
    sa_ref[...] += jnp.sum(a, axis=0, keepdims=True)
    ga_ref[...] += lax.dot_general(a, a, (((0,), (0,)), ((), ())), precision=HP)
    hn_ref[...] = lax.dot_general(xn_ref[...], mn_ref[...], (((1,), (0,)), ((), ())),
                                  precision=DP) + cn_ref[...]


def _msg_body(hs_ref, a_ref, rm_ref, tm_ref, b2_ref, d_ref, o_ref):
    hs = hs_ref[...]
    av = a_ref[...]
    rep = lax.dot_general(hs, rm_ref[...], (((1,), (0,)), ((), ())), precision=DP)
    til = lax.dot_general(av, tm_ref[...], (((1,), (0,)), ((), ())), precision=DP)
    op = rep * til
    msg = lax.dot_general(op, b2_ref[...], (((1,), (0,)), ((), ())), precision=DP)
    msg += lax.dot_general(hs, d_ref[...], (((1,), (0,)), ((), ())), precision=DP)
    o_ref[...] = msg


def _gru_body(a0_ref, a1_ref, d0_ref, d1_ref, h_ref, wih_ref, whh_ref,
              bih_ref, bhh_ref, o_ref):
    agg = a0_ref[...] + a1_ref[...]
    deg = d0_ref[:, 0:1] + d1_ref[:, 0:1]
    deg = jnp.maximum(deg, 1.0)
    x = agg / deg
    h = h_ref[...]
    gi = lax.dot_general(x, wih_ref[...], (((1,), (1,)), ((), ())),
                         precision=DP) + bih_ref[...]
    gh = lax.dot_general(h, whh_ref[...], (((1,), (1,)), ((), ())),
                         precision=DP) + bhh_ref[...]
    r = jax.nn.sigmoid(gi[:, 0:H] + gh[:, 0:H])
    z = jax.nn.sigmoid(gi[:, H:2 * H] + gh[:, H:2 * H])
    n = jnp.tanh(gi[:, 2 * H:3 * H] + r * gh[:, 2 * H:3 * H])
    o_ref[...] = (1.0 - z) * n + z * h


def _s2s_body(h_ref, n2g_ref, wih0_ref, wihr_ref, whh_ref, bih_ref, bhh_ref,
              bng_ref, bnb_ref, c1w_ref, c1b_ref, c2w_ref, c2b_ref, o_ref):
    h = h_ref[...]
    n2g = n2g_ref[...]
    gids = lax.broadcasted_iota(jnp.int32, (NG, N), 0)
    maskb = n2g == gids

    def step(t, carry):
        qs, hstack, cstack = carry
        inp = qs
        new_h = []
        new_c = []
        for l in range(4):
            wih = wih0_ref[...] if l == 0 else wihr_ref[(l - 1) * NG:l * NG, :]
            gates = (lax.dot_general(inp, wih, (((1,), (1,)), ((), ())),
                                     precision=HP)
                     + bih_ref[l:l + 1, :]
                     + lax.dot_general(hstack[l * NG:(l + 1) * NG, :],
                                       whh_ref[l * NG:(l + 1) * NG, :],
                                       (((1,), (1,)), ((), ())), precision=HP)
                     + bhh_ref[l:l + 1, :])
            gi_ = gates[:, 0:H]
            gf_ = gates[:, H:2 * H]
            gg_ = gates[:, 2 * H:3 * H]
            go_ = gates[:, 3 * H:4 * H]
            c = (jax.nn.sigmoid(gf_) * cstack[l * NG:(l + 1) * NG, :]
                 + jax.nn.sigmoid(gi_) * jnp.tanh(gg_))
            hc = jax.nn.sigmoid(go_) * jnp.tanh(c)
            new_h.append(hc)
            new_c.append(c)
            inp = hc
        q = inp
        emat = lax.dot_general(q, h, (((1,), (1,)), ((), ())), precision=HP)
        masked = jnp.where(maskb, emat, -1e30)
        emax = jnp.max(masked, axis=1, keepdims=True)
        ee = jnp.where(maskb, jnp.exp(masked - emax), 0.0)
        denom = jnp.maximum(jnp.sum(ee, axis=1, keepdims=True), 1e-30)
        alpha = ee / denom
        readout = lax.dot_general(alpha, h, (((1,), (0,)), ((), ())),
                                  precision=HP)
        return (jnp.concatenate([q, readout], axis=1),
                jnp.concatenate(new_h, axis=0), jnp.concatenate(new_c, axis=0))

    qs, _, _ = lax.fori_loop(
        0, 6, step,
        (jnp.zeros((NG, 2 * H), jnp.float32),
         jnp.zeros((4 * NG, H), jnp.float32),
         jnp.zeros((4 * NG, H), jnp.float32)))

    m = jnp.mean(qs, axis=0, keepdims=True)
    v = jnp.mean((qs - m) * (qs - m), axis=0, keepdims=True)
    qn_ = (qs - m) * lax.rsqrt(v + 1e-5) * bng_ref[...] + bnb_ref[...]
    o1 = lax.dot_general(qn_, c1w_ref[...], (((1,), (1,)), ((), ())),
                         precision=HP) + c1b_ref[...]
    o1 = jnp.where(o1 >= 0, o1, 0.1 * o1)
    o2 = jnp.sum(o1 * c2w_ref[...], axis=1, keepdims=True) + c2b_ref[...]
    o_ref[...] = jax.nn.sigmoid(o2)


_sc_cache = {}


def _sc_kernels():
    if _sc_cache:
        return _sc_cache
    mesh = plsc.VectorSubcoreMesh(core_axis_name="c", subcore_axis_name="s",
                                  num_cores=NC, num_subcores=NS)
    cparams = pltpu.CompilerParams(use_tc_tiling_on_sc=False)

    @functools.partial(
        pl.kernel,
        mesh=mesh,
        compiler_params=cparams,
        out_type=jax.ShapeDtypeStruct((EPAD, H), jnp.float32),
        scratch_types=[
            pltpu.VMEM((NCHUNK, CHUNK), jnp.int32),
            pltpu.VMEM((EPW, H), jnp.float32),
            pltpu.SemaphoreType.DMA,
        ],
    )
    def sc_gather(h_hbm, src_hbm, out_hbm, idx_v, rows_v, sem):
        wid = lax.axis_index("s") * NC + lax.axis_index("c")
        base = wid * EPW
        pltpu.sync_copy(src_hbm.at[wid], idx_v)
        for g in range(0, NCHUNK, 20):
            fires = [
                pltpu.async_copy(h_hbm.at[idx_v.at[c]],
                                 rows_v.at[pl.ds(c * CHUNK, CHUNK)], sem)
                for c in range(g, g + 20)
            ]
            for d in fires:
                d.wait()
        pltpu.sync_copy(rows_v, out_hbm.at[pl.ds(base, EPW)])

    @functools.partial(
        pl.kernel,
        mesh=mesh,
        compiler_params=cparams,
        out_type=jax.ShapeDtypeStruct((NC * NPAD, H), jnp.float32),
        scratch_types=[
            pltpu.VMEM((NCHUNK, CHUNK), jnp.int32),
            pltpu.VMEM((EPW, H), jnp.float32),
            pltpu.VMEM((STRIPE, H), jnp.float32),
            pltpu.SemaphoreType.DMA,
            pltpu.VMEM_SHARED((NPAD, H), jnp.float32),
        ],
    )
    def sc_scatter(msg_hbm, dst_hbm, out_hbm, idx_v, rows_v, zbuf, sem, shared):
        cid = lax.axis_index("c")
        sid = lax.axis_index("s")
        wid = sid * NC + cid
        base = wid * EPW

        def zr(i, carry):
            zbuf[i, :] = jnp.zeros((H,), jnp.float32)
            return carry

        lax.fori_loop(0, STRIPE, zr, 0)
        pltpu.sync_copy(zbuf, shared.at[pl.ds(sid * STRIPE, STRIPE)])
        pltpu.sync_copy(dst_hbm.at[wid], idx_v)
        pltpu.sync_copy(msg_hbm.at[pl.ds(base, EPW)], rows_v)
        plsc.subcore_barrier()
        for g in range(0, NCHUNK, 20):
            fires = [
                pltpu.async_copy(rows_v.at[pl.ds(c * CHUNK, CHUNK)],
                                 shared.at[idx_v.at[c]], sem, add=True)
                for c in range(g, g + 20)
            ]
            for d in fires:
                d.wait()
        plsc.subcore_barrier()
        pltpu.sync_copy(shared.at[pl.ds(sid * STRIPE, STRIPE)],
                        out_hbm.at[pl.ds(cid * NPAD + sid * STRIPE, STRIPE)])

    @functools.partial(
        pl.kernel,
        mesh=mesh,
        compiler_params=cparams,
        out_type=[jax.ShapeDtypeStruct((NC * NPAD, H), jnp.float32),
                  jax.ShapeDtypeStruct((NC * NPAD, H), jnp.float32)],
        scratch_types=[
            pltpu.VMEM((NCHUNK, CHUNK), jnp.int32),
            pltpu.VMEM((EPW, H), jnp.float32),
            pltpu.VMEM((CHUNK, H), jnp.float32),
            pltpu.VMEM((STRIPE, H), jnp.float32),
            pltpu.SemaphoreType.DMA,
            pltpu.VMEM_SHARED((NPAD, H), jnp.float32),
            pltpu.VMEM_SHARED((NPAD, H), jnp.float32),
        ],
    )
    def sc_scatter_deg(msg_hbm, dst_hbm, agg_hbm, deg_hbm, idx_v, rows_v,
                       ones_v, zbuf, sem, shared_m, shared_d):
        cid = lax.axis_index("c")
        sid = lax.axis_index("s")
        wid = sid * NC + cid
        base = wid * EPW

        def zr(i, carry):
            zbuf[i, :] = jnp.zeros((H,), jnp.float32)
            return carry

        lax.fori_loop(0, STRIPE, zr, 0)

        def onr(i, carry):
            ones_v[i, :] = jnp.ones((H,), jnp.float32)
            return carry

        lax.fori_loop(0, CHUNK, onr, 0)
        pltpu.sync_copy(zbuf, shared_m.at[pl.ds(sid * STRIPE, STRIPE)])
        pltpu.sync_copy(zbuf, shared_d.at[pl.ds(sid * STRIPE, STRIPE)])
        pltpu.sync_copy(dst_hbm.at[wid], idx_v)
        pltpu.sync_copy(msg_hbm.at[pl.ds(base, EPW)], rows_v)
        plsc.subcore_barrier()
        for g in range(0, NCHUNK, 10):
            fires = []
            for c in range(g, g + 10):
                fires.append(
                    pltpu.async_copy(rows_v.at[pl.ds(c * CHUNK, CHUNK)],
                                     shared_m.at[idx_v.at[c]], sem, add=True))
                fires.append(
                    pltpu.async_copy(ones_v, shared_d.at[idx_v.at[c]], sem,
                                     add=True))
            for d in fires:
                d.wait()
        plsc.subcore_barrier()
        pltpu.sync_copy(shared_m.at[pl.ds(sid * STRIPE, STRIPE)],
                        agg_hbm.at[pl.ds(cid * NPAD + sid * STRIPE, STRIPE)])
        pltpu.sync_copy(shared_d.at[pl.ds(sid * STRIPE, STRIPE)],
                        deg_hbm.at[pl.ds(cid * NPAD + sid * STRIPE, STRIPE)])

    _sc_cache.update(gather=sc_gather, scatter=sc_scatter,
                     scatter_deg=sc_scatter_deg)
    return _sc_cache


def _stats(x_edge, x_node):
    nt = 25
    tbe = E // nt
    tbn = N // nt
    return pl.pallas_call(
        _stats_body,
        grid=(nt,),
        in_specs=[pl.BlockSpec((tbe, H), lambda i: (i, 0)),
                  pl.BlockSpec((tbn, NA), lambda i: (i, 0))],
        out_specs=[pl.BlockSpec((1, H), lambda i: (0, 0)),
                   pl.BlockSpec((H, H), lambda i: (0, 0)),
                   pl.BlockSpec((1, NA), lambda i: (0, 0)),
                   pl.BlockSpec((1, NA), lambda i: (0, 0))],
        out_shape=[jax.ShapeDtypeStruct((1, H), jnp.float32),
                   jax.ShapeDtypeStruct((H, H), jnp.float32),
                   jax.ShapeDtypeStruct((1, NA), jnp.float32),
                   jax.ShapeDtypeStruct((1, NA), jnp.float32)],
    )(x_edge, x_node)


def _a_hn(x_edge, m2, c2, x_node, mn, cn):
    nt = 25
    tbe = E // nt
    tbn = N // nt
    return pl.pallas_call(
        _a_hn_body,
        grid=(nt,),
        in_specs=[pl.BlockSpec((tbe, H), lambda i: (i, 0)),
                  pl.BlockSpec((H, H), lambda i: (0, 0)),
                  pl.BlockSpec((1, H), lambda i: (0, 0)),
                  pl.BlockSpec((tbn, NA), lambda i: (i, 0)),
                  pl.BlockSpec((NA, H), lambda i: (0, 0)),
                  pl.BlockSpec((1, H), lambda i: (0, 0))],
        out_specs=[pl.BlockSpec((tbe, H), lambda i: (i, 0)),
                   pl.BlockSpec((1, H), lambda i: (0, 0)),
                   pl.BlockSpec((H, H), lambda i: (0, 0)),
                   pl.BlockSpec((tbn, H), lambda i: (i, 0))],
        out_shape=[jax.ShapeDtypeStruct((E, H), jnp.float32),
                   jax.ShapeDtypeStruct((1, H), jnp.float32),
                   jax.ShapeDtypeStruct((H, H), jnp.float32),
                   jax.ShapeDtypeStruct((N, H), jnp.float32)],
    )(x_edge, m2, c2, x_node, mn, cn)


def _edge_msg(hsrc, a_unpadded, rmat, tmat, b2, dmat):
    nt = 40
    tb = EPAD // nt
    return pl.pallas_call(
        _msg_body,
        grid=(nt,),
        in_specs=[pl.BlockSpec((tb, H), lambda i: (i, 0)),
                  pl.BlockSpec((tb, H), lambda i: (i, 0)),
                  pl.BlockSpec((H, H * H), lambda i: (0, 0)),
                  pl.BlockSpec((H, H * H), lambda i: (0, 0)),
                  pl.BlockSpec((H * H, H), lambda i: (0, 0)),
                  pl.BlockSpec((H, H), lambda i: (0, 0))],
        out_specs=pl.BlockSpec((tb, H), lambda i: (i, 0)),
        out_shape=jax.ShapeDtypeStruct((EPAD, H), jnp.float32),
    )(hsrc, a_unpadded, rmat, tmat, b2, dmat)


def _gru(aggp, degp, h, wih, whh, bih, bhh):
    nt = 10
    tb = N // nt
    row = pl.BlockSpec((tb, H), lambda i: (i, 0))
    full = lambda s: pl.BlockSpec(s, lambda i: (0, 0))
    return pl.pallas_call(
        _gru_body,
        grid=(nt,),
        in_specs=[row, row, row, row, row,
                  full((3 * H, H)), full((3 * H, H)),
                  full((1, 3 * H)), full((1, 3 * H))],
        out_specs=row,
        out_shape=jax.ShapeDtypeStruct((N, H), jnp.float32),
    )(aggp[0:N], aggp[NPAD:NPAD + N], degp[0:N], degp[NPAD:NPAD + N],
      h, wih, whh, bih, bhh)


def _set2set(h, n2g, p):
    return pl.pallas_call(
        _s2s_body,
        out_shape=jax.ShapeDtypeStruct((NG, 1), jnp.float32),
    )(h, n2g,
      p['lstm_Wih0'],
      p['lstm_Wih_rest'].reshape(3 * 4 * H, H),
      p['lstm_Whh'].reshape(4 * 4 * H, H),
      p['lstm_bih'], p['lstm_bhh'],
      p['bn_o_g'].reshape(1, 2 * H), p['bn_o_b'].reshape(1, 2 * H),
      p['c1_W'], p['c1_b'].reshape(1, H),
      p['c2_W'].reshape(1, H), jnp.broadcast_to(p['c2_b'].reshape(1, 1), (NG, 1)))


def kernel(x_node, x_edge, edge_index, node2graph, params):
    p = params
    src = edge_index[0]
    dst = edge_index[1]

    s1e, s2e, s1n, ssn = _stats(x_edge, x_node)

    ef = float(E)
    mu_x = s1e[0] / ef
    cov = s2e / ef - jnp.outer(mu_x, mu_x)
    var_x = jnp.diag(cov)
    se = p['bn_e_g'] * lax.rsqrt(var_x + 1e-5)
    c0 = p['bn_e_b'] - mu_x * se
    m_he = se[:, None] * p['eemb_W'].T
    c_he = c0 @ p['eemb_W'].T + p['eemb_b']
    m1 = m_he @ p['en1_W'].T
    c1v = c_he @ p['en1_W'].T + p['en1_b']
    mean1 = mu_x @ m1 + c1v
    var1 = jnp.sum(m1 * (cov @ m1), axis=0)
    s1 = p['enbn1_g'] * lax.rsqrt(var1 + 1e-5)
    m2 = m1 * s1[None, :]
    c2v = ((c1v - mean1) * s1 + p['enbn1_b']).reshape(1, H)

    mu_n = s1n[0] / float(N)
    var_n = ssn[0] / float(N) - mu_n * mu_n
    sn = p['bn_n_g'] * lax.rsqrt(var_n + 1e-5)
    mn = sn[:, None] * p['nemb_W'].T
    cn = ((p['bn_n_b'] - mu_n * sn) @ p['nemb_W'].T + p['nemb_b']).reshape(1, H)

    a, sa, ga, hn = _a_hn(x_edge, m2, c2v, x_node, mn, cn)

    mean_a = sa[0] / ef
    cov_a = ga / ef - jnp.outer(mean_a, mean_a)
    mean2 = mean_a @ p['en2_W'].T + p['en2_b']
    var2 = jnp.sum((p['en2_W'] @ cov_a) * p['en2_W'], axis=1)
    s2 = p['enbn2_g'] * lax.rsqrt(var2 + 1e-5)
    t2 = p['enbn2_b'] - mean2 * s2
    w3 = p['en2_W'].reshape(H, H, H)
    ahat = w3 * s2.reshape(H, H)[:, :, None]
    b2 = jnp.transpose(ahat, (0, 2, 1)).reshape(H * H, H)
    dmat = (s2 * p['en2_b'] + t2).reshape(H, H)
    k_ids = jnp.arange(H * H, dtype=jnp.int32)
    rows = jnp.arange(H, dtype=jnp.int32)
    rmat = (k_ids[None, :] // H == rows[:, None]).astype(jnp.float32)
    tmat = (k_ids[None, :] % H == rows[:, None]).astype(jnp.float32)

    src_p = jnp.pad(src, (0, EPAD - E)).reshape(NW, NCHUNK, CHUNK)
    dst_p = jnp.pad(dst, (0, EPAD - E), constant_values=DUMMY).reshape(NW, NCHUNK, CHUNK)

    sc = _sc_kernels()

    wih = p['gru_Wih']
    whh = p['gru_Whh']
    bih = p['gru_bih'].reshape(1, 3 * H)
    bhh = p['gru_bhh'].reshape(1, 3 * H)

    hsrc = sc['gather'](hn, src_p)
    msg = _edge_msg(hsrc, a, rmat, tmat, b2, dmat)
    aggp, degp = sc['scatter_deg'](msg, dst_p)
    h = _gru(aggp, degp, hn, wih, whh, bih, bhh)

    hsrc = sc['gather'](h, src_p)
    msg = _edge_msg(hsrc, a, rmat, tmat, b2, dmat)
    aggp = sc['scatter'](msg, dst_p)
    h = _gru(aggp, degp, h, wih, whh, bih, bhh)

    return _set2set(h, node2graph.reshape(1, N), p)
